# baseline (speedup 1.0000x reference)
import jax
import jax.numpy as jnp
from jax import lax
from jax.experimental import pallas as pl
from jax.experimental.pallas import tpu as pltpu
from jax.experimental.pallas import tpu_sc as plsc

D = 32
_NC = 2
_NS = 16
_NW = _NC * _NS
_RPC = 4
_IPC = _RPC * 26
_CPW = 128

_V = 1000000
_TR_FULL = _V // 128
_TAIL = _V - _TR_FULL * 128


def _transpose_chunk(in_ref, out_ref, nrow):
    ii = lax.iota(jnp.int32, 16)

    def vrow_step(vrow, carry):
        for h in range(8):
            c_idx = ii + 16 * (h % 2)
            i_scalar = 4 * vrow + (h // 2)
            i_idx = jnp.zeros((16,), jnp.int32) + i_scalar
            val = plsc.load_gather(in_ref, [c_idx, i_idx])
            out_ref[vrow, pl.ds(16 * h, 16)] = val
        return carry

    lax.fori_loop(0, nrow, vrow_step, 0)


def _relayout_body(tabt_hbm, tail_hbm, out_hbm, in_v, out_v, si0, si1, so0, so1):
    wid = lax.axis_index("s") * _NC + lax.axis_index("c")
    sems_in = (si0, si1)
    sems_out = (so0, so1)
    extra = jnp.where(wid < 4, 1, 0)
    n_chunks = 244 + extra
    base = wid * 244 + jnp.minimum(wid, 4)

    def start_in(k, b):
        tr = base + k
        pltpu.async_copy(
            tabt_hbm.at[:, pl.ds(pl.multiple_of(tr * 128, 128), 128)],
            in_v.at[b],
            sems_in[b],
        )

    def wait_in(b):
        pltpu.make_async_copy(
            tabt_hbm.at[:, pl.ds(0, 128)], in_v.at[b], sems_in[b]
        ).wait()

    def start_out(k, b):
        tr = base + k
        pltpu.async_copy(
            out_v.at[b],
            out_hbm.at[pl.ds(pl.multiple_of(tr * 32, 32), 32)],
            sems_out[b],
        )

    def drain_out(b):
        pltpu.make_async_copy(
            out_v.at[b], out_hbm.at[pl.ds(0, 32)], sems_out[b]
        ).wait()

    start_in(0, 0)

    def step(g, carry):
        for b in range(2):
            k = 2 * g + b

            @pl.when(k < n_chunks)
            def _():
                wait_in(b)

                @pl.when(k + 1 < n_chunks)
                def _():
                    start_in(k + 1, 1 - b)

                @pl.when(k >= 2)
                def _():
                    drain_out(b)

                _transpose_chunk(in_v.at[b], out_v.at[b], 32)
                start_out(k, b)

        return carry

    lax.fori_loop(0, 123, step, 0)
    drain_out(0)
    drain_out(1)

    @pl.when(wid == 0)
    def _():
        pltpu.sync_copy(tail_hbm, out_hbm.at[pl.ds(_TR_FULL * 32, 16)])


def _gather_body(table_hbm, idx_hbm, out_hbm, idx_v, rows_v,
                 sem_in0, sem_in1, sem_out0, sem_out1):
    wid = lax.axis_index("s") * _NC + lax.axis_index("c")
    row0 = wid * (_CPW * _RPC)
    sems_in = (sem_in0, sem_in1)
    sems_out = (sem_out0, sem_out1)
    pltpu.sync_copy(idx_hbm.at[wid], idx_v)

    def start_gather(k, b):
        pltpu.async_copy(table_hbm.at[idx_v.at[k]], rows_v.at[b], sems_in[b])

    def wait_gather(b):
        pltpu.make_async_copy(
            table_hbm.at[pl.ds(0, _IPC)], rows_v.at[b], sems_in[b]
        ).wait()

    def start_writes(k, b):
        for m in range(_RPC):
            pltpu.async_copy(
                rows_v.at[b].at[pl.ds(26 * m, 26)],
                out_hbm.at[row0 + k * _RPC + m],
                sems_out[b],
            )

    def drain_writes(b):
        for m in range(_RPC):
            pltpu.make_async_copy(
                rows_v.at[b].at[pl.ds(26 * m, 26)], out_hbm.at[0], sems_out[b]
            ).wait()

    start_gather(0, 0)

    def step(g, carry):
        for b in range(2):
            k = 2 * g + b
            wait_gather(b)
            start_writes(k, b)

            @pl.when(k + 1 < _CPW)
            def _():
                nb = 1 - b

                @pl.when(k >= 1)
                def _():
                    drain_writes(nb)

                start_gather(k + 1, nb)

        return carry

    lax.fori_loop(0, _CPW // 2, step, 0)
    drain_writes(0)
    drain_writes(1)


def kernel(x, table):
    rows, cols = x.shape
    idx = x.astype(jnp.int32).reshape(_NW, _CPW, _IPC)
    mesh = plsc.VectorSubcoreMesh(core_axis_name="c", subcore_axis_name="s")

    relayout = pl.kernel(
        _relayout_body,
        mesh=mesh,
        out_type=jax.ShapeDtypeStruct((_V // 4, 128), jnp.float32),
        scratch_types=[
            pltpu.VMEM((2, 32, 128), jnp.float32),
            pltpu.VMEM((2, 32, 128), jnp.float32),
            pltpu.SemaphoreType.DMA,
            pltpu.SemaphoreType.DMA,
            pltpu.SemaphoreType.DMA,
            pltpu.SemaphoreType.DMA,
        ],
        compiler_params=pltpu.CompilerParams(needs_layout_passes=False),
    )

    gather = pl.kernel(
        _gather_body,
        mesh=mesh,
        out_type=jax.ShapeDtypeStruct((rows, cols, D), jnp.float32),
        scratch_types=[
            pltpu.VMEM((_CPW, _IPC), jnp.int32),
            pltpu.VMEM((2, _IPC, D), jnp.float32),
            pltpu.SemaphoreType.DMA,
            pltpu.SemaphoreType.DMA,
            pltpu.SemaphoreType.DMA,
            pltpu.SemaphoreType.DMA,
        ],
        compiler_params=pltpu.CompilerParams(use_tc_tiling_on_sc=False),
    )

    tail = table[_TR_FULL * 128:].reshape(16, 128)
    table_lin = relayout(table.T, tail).reshape(_V, D)
    return gather(table_lin, idx)
